# Initial kernel scaffold; baseline (speedup 1.0000x reference)
import jax
import jax.numpy as jnp
from jax.experimental import pallas as pl


def kernel(x, edge_index, edge_attr, batch, We, be, W1, b1, W2, b2, eps, gamma, beta):
    raise NotImplementedError("write your pallas kernel here")


# speedup vs baseline: 2.8601x; 2.8601x over previous
import functools

import jax
import jax.numpy as jnp
from jax import lax
from jax.experimental import pallas as pl
from jax.experimental.pallas import tpu as pltpu
from jax.experimental.pallas import tpu_sc as plsc

N = 10000
E = 320000
D = 128
ED = 16

NC = 2
NS = 16

BLK = 128
NBLK = E // BLK
BLK_PER_SC = NBLK // NC
MAX_I = -(-BLK_PER_SC // NS)
N_PAD = 10240
ROWS_PER_TILE = N_PAD // NS
ZROWS = 64


def _emb_body(ea_ref, we_ref, be_ref, out_ref):
    out_ref[...] = (
        jnp.dot(ea_ref[...], we_ref[...], preferred_element_type=jnp.float32)
        + be_ref[...]
    )


def _edge_emb(edge_attr, We, be):
    blk_e = 3200
    grid = (E // blk_e,)
    return pl.pallas_call(
        _emb_body,
        grid=grid,
        in_specs=[
            pl.BlockSpec((blk_e, ED), lambda i: (i, 0)),
            pl.BlockSpec((ED, D), lambda i: (0, 0)),
            pl.BlockSpec((1, D), lambda i: (0, 0)),
        ],
        out_specs=pl.BlockSpec((blk_e, D), lambda i: (i, 0)),
        out_shape=jax.ShapeDtypeStruct((E, D), jnp.float32),
    )(edge_attr, We, be)


def _sc_body(x_hbm, src_hbm, dst_hbm, emb_hbm, out_hbm,
             sidx, didx, emb_v, xrow_v, zbuf, sem, acc_sh):
    c = lax.axis_index("c")
    s = lax.axis_index("s")

    def zero_row(r, _):
        for j in range(D // 16):
            zbuf[r, pl.ds(j * 16, 16)] = jnp.zeros((16,), jnp.float32)
        return 0

    lax.fori_loop(0, ZROWS, zero_row, 0)
    for t in range(ROWS_PER_TILE // ZROWS):
        pltpu.sync_copy(
            zbuf, acc_sh.at[pl.ds(s * ROWS_PER_TILE + t * ZROWS, ZROWS), :]
        )
    plsc.subcore_barrier()

    def blk_body(i, _):
        bid = i * NS + s

        @pl.when(bid < BLK_PER_SC)
        def _():
            base = (c * BLK_PER_SC + bid) * BLK
            pltpu.sync_copy(src_hbm.at[pl.ds(base, BLK)], sidx)
            pltpu.sync_copy(dst_hbm.at[pl.ds(base, BLK)], didx.at[0])
            pltpu.sync_copy(emb_hbm.at[pl.ds(base, BLK), :], emb_v)
            pltpu.async_copy(x_hbm.at[sidx], xrow_v, sem).wait()

            def ew(r, _2):
                for j in range(D // 16):
                    sl = pl.ds(j * 16, 16)
                    emb_v[r, sl] = jnp.maximum(
                        emb_v[r, sl] + xrow_v[r, sl], 0.0
                    )
                return 0

            lax.fori_loop(0, BLK, ew, 0)
            pltpu.sync_copy(emb_v, acc_sh.at[didx.at[0]], add=True)

        return 0

    lax.fori_loop(0, MAX_I, blk_body, 0)
    plsc.subcore_barrier()

    for t in range(ROWS_PER_TILE // ZROWS):
        row0 = s * ROWS_PER_TILE + t * ZROWS
        pltpu.sync_copy(acc_sh.at[pl.ds(row0, ZROWS), :], zbuf)
        pltpu.sync_copy(zbuf, out_hbm.at[c, pl.ds(row0, ZROWS), :])


def _sc_aggregate(x, src, dst, edge_emb):
    mesh = plsc.VectorSubcoreMesh(core_axis_name="c", subcore_axis_name="s")
    k = pl.kernel(
        _sc_body,
        out_type=jax.ShapeDtypeStruct((NC, N_PAD, D), jnp.float32),
        mesh=mesh,
        scratch_types=[
            pltpu.VMEM((BLK,), jnp.int32),
            pltpu.VMEM((1, BLK), jnp.int32),
            pltpu.VMEM((BLK, D), jnp.float32),
            pltpu.VMEM((BLK, D), jnp.float32),
            pltpu.VMEM((ZROWS, D), jnp.float32),
            pltpu.SemaphoreType.DMA,
            pltpu.VMEM_SHARED((N_PAD, D), jnp.float32),
        ],
    )
    return k(x, src, dst, edge_emb)


def _mlp_body(x_ref, p_ref, w1_ref, b1_ref, w2_ref, b2_ref, g_ref, bt_ref,
              eps_ref, out_ref):
    h = x_ref[...] * (1.0 + eps_ref[0, 0]) + p_ref[0] + p_ref[1]
    h1 = jnp.maximum(
        jnp.dot(h, w1_ref[...], preferred_element_type=jnp.float32)
        + b1_ref[...],
        0.0,
    )
    h2 = (
        jnp.dot(h1, w2_ref[...], preferred_element_type=jnp.float32)
        + b2_ref[...]
    )
    mean = jnp.mean(h2, axis=-1, keepdims=True)
    var = jnp.mean((h2 - mean) * (h2 - mean), axis=-1, keepdims=True)
    hn = (h2 - mean) * lax.rsqrt(var + 1e-5) * g_ref[...] + bt_ref[...]
    out_ref[...] = jnp.maximum(hn, 0.0)


def _mlp_ln(x, parts, W1, b1, W2, b2, gamma, beta, eps):
    blk_n = 1000
    grid = (N // blk_n,)
    return pl.pallas_call(
        _mlp_body,
        grid=grid,
        in_specs=[
            pl.BlockSpec((blk_n, D), lambda i: (i, 0)),
            pl.BlockSpec((NC, blk_n, D), lambda i: (0, i, 0)),
            pl.BlockSpec((D, D), lambda i: (0, 0)),
            pl.BlockSpec((1, D), lambda i: (0, 0)),
            pl.BlockSpec((D, D), lambda i: (0, 0)),
            pl.BlockSpec((1, D), lambda i: (0, 0)),
            pl.BlockSpec((1, D), lambda i: (0, 0)),
            pl.BlockSpec((1, D), lambda i: (0, 0)),
            pl.BlockSpec((1, 1), lambda i: (0, 0)),
        ],
        out_specs=pl.BlockSpec((blk_n, D), lambda i: (i, 0)),
        out_shape=jax.ShapeDtypeStruct((N, D), jnp.float32),
    )(x, parts, W1, b1, W2, b2, gamma, beta, eps)


def kernel(x, edge_index, edge_attr, batch, We, be, W1, b1, W2, b2, eps,
           gamma, beta):
    del batch
    edge_emb = _edge_emb(edge_attr, We, be.reshape(1, D))
    src = edge_index[0]
    dst = edge_index[1]
    parts = _sc_aggregate(x, src, dst, edge_emb)
    return _mlp_ln(
        x, parts, W1, b1.reshape(1, D), W2, b2.reshape(1, D),
        gamma.reshape(1, D), beta.reshape(1, D), eps.reshape(1, 1),
    )


# speedup vs baseline: 3.5423x; 1.2385x over previous
import jax
import jax.numpy as jnp
from jax import lax
from jax.experimental import pallas as pl
from jax.experimental.pallas import tpu as pltpu
from jax.experimental.pallas import tpu_sc as plsc

N = 10000
E = 320000
D = 128
ED = 16

NC = 2
NS = 16

BLK = 64
NBLK = E // BLK
BLK_PER_SC = NBLK // NC
NPAIR = -(-(-(-BLK_PER_SC // NS)) // 2)
N_PAD = 10240
ROWS_PER_TILE = N_PAD // NS


def _emb_body(ea_ref, we_ref, be_ref, out_ref):
    out_ref[...] = (
        jnp.dot(ea_ref[...], we_ref[...], preferred_element_type=jnp.float32)
        + be_ref[...]
    )


def _edge_emb(edge_attr, We, be):
    blk_e = 3200
    grid = (E // blk_e,)
    return pl.pallas_call(
        _emb_body,
        grid=grid,
        in_specs=[
            pl.BlockSpec((blk_e, ED), lambda i: (i, 0)),
            pl.BlockSpec((ED, D), lambda i: (0, 0)),
            pl.BlockSpec((1, D), lambda i: (0, 0)),
        ],
        out_specs=pl.BlockSpec((blk_e, D), lambda i: (i, 0)),
        out_shape=jax.ShapeDtypeStruct((E, D), jnp.float32),
    )(edge_attr, We, be)


def _sc_body(x_hbm, src_hbm, dst_hbm, emb_hbm, out_hbm,
             sidx, didx, emb_v, xrow_v, sem_e0, sem_e1, sem_g0, sem_g1,
             acc_sh):
    c = lax.axis_index("c")
    s = lax.axis_index("s")
    sems_e = (sem_e0, sem_e1)
    sems_g = (sem_g0, sem_g1)

    def zero_row(r, _):
        for j in range(D // 16):
            emb_v[0, r, pl.ds(j * 16, 16)] = jnp.zeros((16,), jnp.float32)
        return 0

    lax.fori_loop(0, BLK, zero_row, 0)
    for t in range(ROWS_PER_TILE // BLK):
        pltpu.sync_copy(
            emb_v.at[0],
            acc_sh.at[pl.ds(s * ROWS_PER_TILE + t * BLK, BLK), :],
        )
    plsc.subcore_barrier()

    def issue(j, b):
        base = (c * BLK_PER_SC + j * NS + s) * BLK
        pltpu.sync_copy(src_hbm.at[pl.ds(base, BLK)], sidx.at[b])
        pltpu.sync_copy(dst_hbm.at[pl.ds(base, BLK)], didx.at[b, 0])
        pltpu.async_copy(emb_hbm.at[pl.ds(base, BLK), :], emb_v.at[b],
                         sems_e[b])
        pltpu.async_copy(x_hbm.at[sidx.at[b]], xrow_v.at[b], sems_g[b])

    issue(0, 0)
    issue(1, 1)

    def pair(jj, _):
        for b in range(2):
            j = 2 * jj + b

            @pl.when(j * NS + s < BLK_PER_SC)
            def _():
                pltpu.make_async_copy(
                    emb_hbm.at[pl.ds(0, BLK), :], emb_v.at[b], sems_e[b]
                ).wait()
                pltpu.make_async_copy(
                    x_hbm.at[sidx.at[b]], xrow_v.at[b], sems_g[b]
                ).wait()

                def ew(r, _2):
                    for g in range(D // 16):
                        sl = pl.ds(g * 16, 16)
                        emb_v[b, r, sl] = jnp.maximum(
                            emb_v[b, r, sl] + xrow_v[b, r, sl], 0.0
                        )
                    return 0

                lax.fori_loop(0, BLK, ew, 0)
                pltpu.sync_copy(emb_v.at[b], acc_sh.at[didx.at[b, 0]],
                                add=True)

            @pl.when((j + 2) * NS + s < BLK_PER_SC)
            def _():
                issue(j + 2, b)

        return 0

    lax.fori_loop(0, NPAIR, pair, 0)
    plsc.subcore_barrier()

    for t in range(ROWS_PER_TILE // BLK):
        row0 = s * ROWS_PER_TILE + t * BLK
        pltpu.sync_copy(acc_sh.at[pl.ds(row0, BLK), :], emb_v.at[0])
        pltpu.sync_copy(emb_v.at[0], out_hbm.at[c, pl.ds(row0, BLK), :])


def _sc_aggregate(x, src, dst, edge_emb):
    mesh = plsc.VectorSubcoreMesh(core_axis_name="c", subcore_axis_name="s")
    k = pl.kernel(
        _sc_body,
        out_type=jax.ShapeDtypeStruct((NC, N_PAD, D), jnp.float32),
        mesh=mesh,
        scratch_types=[
            pltpu.VMEM((2, BLK), jnp.int32),
            pltpu.VMEM((2, 1, BLK), jnp.int32),
            pltpu.VMEM((2, BLK, D), jnp.float32),
            pltpu.VMEM((2, BLK, D), jnp.float32),
            pltpu.SemaphoreType.DMA,
            pltpu.SemaphoreType.DMA,
            pltpu.SemaphoreType.DMA,
            pltpu.SemaphoreType.DMA,
            pltpu.VMEM_SHARED((N_PAD, D), jnp.float32),
        ],
    )
    return k(x, src, dst, edge_emb)


def _mlp_body(x_ref, p_ref, w1_ref, b1_ref, w2_ref, b2_ref, g_ref, bt_ref,
              eps_ref, out_ref):
    h = x_ref[...] * (1.0 + eps_ref[0, 0]) + p_ref[0] + p_ref[1]
    h1 = jnp.maximum(
        jnp.dot(h, w1_ref[...], preferred_element_type=jnp.float32)
        + b1_ref[...],
        0.0,
    )
    h2 = (
        jnp.dot(h1, w2_ref[...], preferred_element_type=jnp.float32)
        + b2_ref[...]
    )
    mean = jnp.mean(h2, axis=-1, keepdims=True)
    var = jnp.mean((h2 - mean) * (h2 - mean), axis=-1, keepdims=True)
    hn = (h2 - mean) * lax.rsqrt(var + 1e-5) * g_ref[...] + bt_ref[...]
    out_ref[...] = jnp.maximum(hn, 0.0)


def _mlp_ln(x, parts, W1, b1, W2, b2, gamma, beta, eps):
    blk_n = 1000
    grid = (N // blk_n,)
    return pl.pallas_call(
        _mlp_body,
        grid=grid,
        in_specs=[
            pl.BlockSpec((blk_n, D), lambda i: (i, 0)),
            pl.BlockSpec((NC, blk_n, D), lambda i: (0, i, 0)),
            pl.BlockSpec((D, D), lambda i: (0, 0)),
            pl.BlockSpec((1, D), lambda i: (0, 0)),
            pl.BlockSpec((D, D), lambda i: (0, 0)),
            pl.BlockSpec((1, D), lambda i: (0, 0)),
            pl.BlockSpec((1, D), lambda i: (0, 0)),
            pl.BlockSpec((1, D), lambda i: (0, 0)),
            pl.BlockSpec((1, 1), lambda i: (0, 0)),
        ],
        out_specs=pl.BlockSpec((blk_n, D), lambda i: (i, 0)),
        out_shape=jax.ShapeDtypeStruct((N, D), jnp.float32),
    )(x, parts, W1, b1, W2, b2, gamma, beta, eps)


def kernel(x, edge_index, edge_attr, batch, We, be, W1, b1, W2, b2, eps,
           gamma, beta):
    del batch
    edge_emb = _edge_emb(edge_attr, We, be.reshape(1, D))
    src = edge_index[0]
    dst = edge_index[1]
    parts = _sc_aggregate(x, src, dst, edge_emb)
    return _mlp_ln(
        x, parts, W1, b1.reshape(1, D), W2, b2.reshape(1, D),
        gamma.reshape(1, D), beta.reshape(1, D), eps.reshape(1, 1),
    )


# speedup vs baseline: 4.3766x; 1.2355x over previous
import jax
import jax.numpy as jnp
from jax import lax
from jax.experimental import pallas as pl
from jax.experimental.pallas import tpu as pltpu
from jax.experimental.pallas import tpu_sc as plsc

N = 10000
E = 320000
D = 128
ED = 16

NC = 2
NS = 16

BLK = 64
NBLK = E // BLK
BLK_PER_SC = NBLK // NC
NPAIR = -(-(-(-BLK_PER_SC // NS)) // 2)
N_PAD = 10240
ROWS_PER_TILE = N_PAD // NS


def _emb_body(ea_ref, we_ref, be_ref, out_ref):
    out_ref[...] = (
        lax.dot_general(
            ea_ref[...], we_ref[...], (((0,), (0,)), ((), ())),
            preferred_element_type=jnp.float32,
        )
        + be_ref[...]
    )


def _edge_emb(edge_attr_t, We, be):
    blk_e = 3200
    grid = (E // blk_e,)
    return pl.pallas_call(
        _emb_body,
        grid=grid,
        in_specs=[
            pl.BlockSpec((ED, blk_e), lambda i: (0, i)),
            pl.BlockSpec((ED, D), lambda i: (0, 0)),
            pl.BlockSpec((1, D), lambda i: (0, 0)),
        ],
        out_specs=pl.BlockSpec((blk_e, D), lambda i: (i, 0)),
        out_shape=jax.ShapeDtypeStruct((E, D), jnp.float32),
    )(edge_attr_t, We, be)


def _sc_body(x_hbm, src_hbm, dst_hbm, emb_hbm, out_hbm,
             sidx, didx, emb_v, xrow_v, sem_e0, sem_e1, sem_g0, sem_g1,
             acc_sh):
    c = lax.axis_index("c")
    s = lax.axis_index("s")
    sems_e = (sem_e0, sem_e1)
    sems_g = (sem_g0, sem_g1)

    def zero_row(r, _):
        for j in range(D // 16):
            emb_v[0, r, pl.ds(j * 16, 16)] = jnp.zeros((16,), jnp.float32)
        return 0

    lax.fori_loop(0, BLK, zero_row, 0)
    for t in range(ROWS_PER_TILE // BLK):
        pltpu.sync_copy(
            emb_v.at[0],
            acc_sh.at[pl.ds(s * ROWS_PER_TILE + t * BLK, BLK), :],
        )
    plsc.subcore_barrier()

    def issue(j, b):
        base = (c * BLK_PER_SC + j * NS + s) * BLK
        pltpu.sync_copy(src_hbm.at[pl.ds(base, BLK)], sidx.at[b])
        pltpu.sync_copy(dst_hbm.at[pl.ds(base, BLK)], didx.at[b, 0])
        pltpu.async_copy(emb_hbm.at[pl.ds(base, BLK), :], emb_v.at[b],
                         sems_e[b])
        pltpu.async_copy(x_hbm.at[sidx.at[b]], xrow_v.at[b], sems_g[b])

    issue(0, 0)
    issue(1, 1)

    def pair(jj, _):
        for b in range(2):
            j = 2 * jj + b

            @pl.when(j * NS + s < BLK_PER_SC)
            def _():
                pltpu.make_async_copy(
                    emb_hbm.at[pl.ds(0, BLK), :], emb_v.at[b], sems_e[b]
                ).wait()
                pltpu.make_async_copy(
                    x_hbm.at[sidx.at[b]], xrow_v.at[b], sems_g[b]
                ).wait()

                def ew(r2, _2):
                    for u in range(4):
                        r = 4 * r2 + u
                        for g in range(D // 16):
                            sl = pl.ds(g * 16, 16)
                            emb_v[b, r, sl] = jnp.maximum(
                                emb_v[b, r, sl] + xrow_v[b, r, sl], 0.0
                            )
                    return 0

                lax.fori_loop(0, BLK // 4, ew, 0)
                pltpu.sync_copy(emb_v.at[b], acc_sh.at[didx.at[b, 0]],
                                add=True)

            @pl.when((j + 2) * NS + s < BLK_PER_SC)
            def _():
                issue(j + 2, b)

        return 0

    lax.fori_loop(0, NPAIR, pair, 0)
    plsc.subcore_barrier()

    for t in range(ROWS_PER_TILE // BLK):
        row0 = s * ROWS_PER_TILE + t * BLK
        pltpu.sync_copy(acc_sh.at[pl.ds(row0, BLK), :], emb_v.at[0])
        pltpu.sync_copy(emb_v.at[0], out_hbm.at[c, pl.ds(row0, BLK), :])


def _sc_aggregate(x, src, dst, edge_emb):
    mesh = plsc.VectorSubcoreMesh(core_axis_name="c", subcore_axis_name="s")
    k = pl.kernel(
        _sc_body,
        out_type=jax.ShapeDtypeStruct((NC, N_PAD, D), jnp.float32),
        mesh=mesh,
        scratch_types=[
            pltpu.VMEM((2, BLK), jnp.int32),
            pltpu.VMEM((2, 1, BLK), jnp.int32),
            pltpu.VMEM((2, BLK, D), jnp.float32),
            pltpu.VMEM((2, BLK, D), jnp.float32),
            pltpu.SemaphoreType.DMA,
            pltpu.SemaphoreType.DMA,
            pltpu.SemaphoreType.DMA,
            pltpu.SemaphoreType.DMA,
            pltpu.VMEM_SHARED((N_PAD, D), jnp.float32),
        ],
    )
    return k(x, src, dst, edge_emb)


def _mlp_body(x_ref, p_ref, w1_ref, b1_ref, w2_ref, b2_ref, g_ref, bt_ref,
              eps_ref, out_ref):
    h = x_ref[...] * (1.0 + eps_ref[0, 0]) + p_ref[0] + p_ref[1]
    h1 = jnp.maximum(
        jnp.dot(h, w1_ref[...], preferred_element_type=jnp.float32)
        + b1_ref[...],
        0.0,
    )
    h2 = (
        jnp.dot(h1, w2_ref[...], preferred_element_type=jnp.float32)
        + b2_ref[...]
    )
    mean = jnp.mean(h2, axis=-1, keepdims=True)
    var = jnp.mean((h2 - mean) * (h2 - mean), axis=-1, keepdims=True)
    hn = (h2 - mean) * lax.rsqrt(var + 1e-5) * g_ref[...] + bt_ref[...]
    out_ref[...] = jnp.maximum(hn, 0.0)


def _mlp_ln(x, parts, W1, b1, W2, b2, gamma, beta, eps):
    blk_n = 1000
    grid = (N // blk_n,)
    return pl.pallas_call(
        _mlp_body,
        grid=grid,
        in_specs=[
            pl.BlockSpec((blk_n, D), lambda i: (i, 0)),
            pl.BlockSpec((NC, blk_n, D), lambda i: (0, i, 0)),
            pl.BlockSpec((D, D), lambda i: (0, 0)),
            pl.BlockSpec((1, D), lambda i: (0, 0)),
            pl.BlockSpec((D, D), lambda i: (0, 0)),
            pl.BlockSpec((1, D), lambda i: (0, 0)),
            pl.BlockSpec((1, D), lambda i: (0, 0)),
            pl.BlockSpec((1, D), lambda i: (0, 0)),
            pl.BlockSpec((1, 1), lambda i: (0, 0)),
        ],
        out_specs=pl.BlockSpec((blk_n, D), lambda i: (i, 0)),
        out_shape=jax.ShapeDtypeStruct((N, D), jnp.float32),
    )(x, parts, W1, b1, W2, b2, gamma, beta, eps)


def kernel(x, edge_index, edge_attr, batch, We, be, W1, b1, W2, b2, eps,
           gamma, beta):
    del batch
    edge_emb = _edge_emb(edge_attr.T, We, be.reshape(1, D))
    src = edge_index[0]
    dst = edge_index[1]
    parts = _sc_aggregate(x, src, dst, edge_emb)
    return _mlp_ln(
        x, parts, W1, b1.reshape(1, D), W2, b2.reshape(1, D),
        gamma.reshape(1, D), beta.reshape(1, D), eps.reshape(1, 1),
    )


# speedup vs baseline: 5.8229x; 1.3305x over previous
import jax
import jax.numpy as jnp
from jax import lax
from jax.experimental import pallas as pl
from jax.experimental.pallas import tpu as pltpu
from jax.experimental.pallas import tpu_sc as plsc

N = 10000
E = 320000
D = 128
ED = 16

NC = 2
NS = 16

BLK = 64
NBLK = E // BLK
BLK_PER_SC = NBLK // NC
MAX_J = -(-BLK_PER_SC // NS)
N_PAD = 10240
ROWS_PER_TILE = N_PAD // NS


def _emb_body(ea_ref, we_ref, be_ref, out_ref):
    out_ref[...] = (
        lax.dot_general(
            ea_ref[...], we_ref[...], (((0,), (0,)), ((), ())),
            preferred_element_type=jnp.float32,
        )
        + be_ref[...]
    )


def _edge_emb(edge_attr_t, We, be):
    blk_e = 3200
    grid = (E // blk_e,)
    return pl.pallas_call(
        _emb_body,
        grid=grid,
        in_specs=[
            pl.BlockSpec((ED, blk_e), lambda i: (0, i)),
            pl.BlockSpec((ED, D), lambda i: (0, 0)),
            pl.BlockSpec((1, D), lambda i: (0, 0)),
        ],
        out_specs=pl.BlockSpec((blk_e, D), lambda i: (i, 0)),
        out_shape=jax.ShapeDtypeStruct((E, D), jnp.float32),
    )(edge_attr_t, We, be)


def _sc_body(x_hbm, src_hbm, dst_hbm, emb_hbm, out_hbm,
             sidx, didx, emb_v, xrow_v, embsem, gsem, scsem, isem, dsem,
             acc_sh):
    c = lax.axis_index("c")
    s = lax.axis_index("s")

    def zero_row(r, _):
        for j in range(D // 16):
            emb_v[0, r, pl.ds(j * 16, 16)] = jnp.zeros((16,), jnp.float32)
        return 0

    lax.fori_loop(0, BLK, zero_row, 0)
    for t in range(ROWS_PER_TILE // BLK):
        pltpu.sync_copy(
            emb_v.at[0],
            acc_sh.at[pl.ds(s * ROWS_PER_TILE + t * BLK, BLK), :],
        )
    plsc.subcore_barrier()

    def valid(j):
        return j * NS + s < BLK_PER_SC

    def base_of(j):
        return (c * BLK_PER_SC + j * NS + s) * BLK

    def issue_idx(j, sl):
        base = base_of(j)
        pltpu.async_copy(src_hbm.at[pl.ds(base, BLK)], sidx.at[sl],
                         isem.at[sl])
        pltpu.async_copy(dst_hbm.at[pl.ds(base, BLK)], didx.at[sl, 0],
                         dsem.at[sl])

    def wait_idx(sl):
        pltpu.make_async_copy(src_hbm.at[pl.ds(0, BLK)], sidx.at[sl],
                              isem.at[sl]).wait()
        pltpu.make_async_copy(dst_hbm.at[pl.ds(0, BLK)], didx.at[sl, 0],
                              dsem.at[sl]).wait()

    def issue_emb(j, e):
        pltpu.async_copy(emb_hbm.at[pl.ds(base_of(j), BLK), :],
                         emb_v.at[e], embsem.at[e])

    def wait_emb(e):
        pltpu.make_async_copy(emb_hbm.at[pl.ds(0, BLK), :], emb_v.at[e],
                              embsem.at[e]).wait()

    def issue_gather(sl, xs):
        pltpu.async_copy(x_hbm.at[sidx.at[sl]], xrow_v.at[xs], gsem.at[xs])

    def wait_gather(xs):
        pltpu.make_async_copy(x_hbm.at[sidx.at[0]], xrow_v.at[xs],
                              gsem.at[xs]).wait()

    def issue_scatter(e, sl):
        pltpu.make_async_copy(emb_v.at[e], acc_sh.at[didx.at[sl, 0]],
                              scsem.at[e]).start(add=True)

    def wait_scatter(e):
        pltpu.make_async_copy(emb_v.at[e], acc_sh.at[didx.at[0, 0]],
                              scsem.at[e]).wait()

    def compute(e, xs):
        def ew(r2, _2):
            for u in range(4):
                r = 4 * r2 + u
                for g in range(D // 16):
                    sl = pl.ds(g * 16, 16)
                    emb_v[e, r, sl] = jnp.maximum(
                        emb_v[e, r, sl] + xrow_v[xs, r, sl], 0.0
                    )
            return 0

        lax.fori_loop(0, BLK // 4, ew, 0)

    for k in range(6):
        issue_idx(k, k)
    for e in range(3):
        issue_emb(e, e)
    for j0 in range(2):
        wait_idx(j0)
        issue_gather(j0, j0)
    for j0 in range(2):
        wait_emb(j0 % 3)
        wait_gather(j0 % 2)
        compute(j0 % 3, j0 % 2)
        issue_scatter(j0 % 3, j0 % 6)
        wait_idx(j0 + 2)
        issue_gather(j0 + 2, j0 % 2)

    def step(q, _):
        for u in range(6):
            j = 2 + 6 * q + u
            e3 = (2 + u) % 3
            x2 = u % 2
            i6 = (2 + u) % 6

            @pl.when(valid(j + 1))
            def _():
                wait_scatter((e3 + 1) % 3)
                issue_emb(j + 1, (e3 + 1) % 3)

            @pl.when(valid(j + 4))
            def _():
                issue_idx(j + 4, (i6 + 4) % 6)

            @pl.when(valid(j))
            def _():
                wait_emb(e3)
                wait_gather(x2)
                compute(e3, x2)
                issue_scatter(e3, i6)

            @pl.when(valid(j + 2))
            def _():
                wait_idx((i6 + 2) % 6)
                issue_gather((i6 + 2) % 6, x2)

        return 0

    lax.fori_loop(0, (MAX_J - 2 + 5) // 6, step, 0)

    for e in range(3):
        wait_scatter(e)
    plsc.subcore_barrier()

    for t in range(ROWS_PER_TILE // BLK):
        row0 = s * ROWS_PER_TILE + t * BLK
        pltpu.sync_copy(acc_sh.at[pl.ds(row0, BLK), :], emb_v.at[0])
        pltpu.sync_copy(emb_v.at[0], out_hbm.at[c, pl.ds(row0, BLK), :])


def _sc_aggregate(x, src, dst, edge_emb):
    mesh = plsc.VectorSubcoreMesh(core_axis_name="c", subcore_axis_name="s")
    k = pl.kernel(
        _sc_body,
        out_type=jax.ShapeDtypeStruct((NC, N_PAD, D), jnp.float32),
        mesh=mesh,
        scratch_types=[
            pltpu.VMEM((6, BLK), jnp.int32),
            pltpu.VMEM((6, 1, BLK), jnp.int32),
            pltpu.VMEM((3, BLK, D), jnp.float32),
            pltpu.VMEM((2, BLK, D), jnp.float32),
            pltpu.SemaphoreType.DMA((3,)),
            pltpu.SemaphoreType.DMA((2,)),
            pltpu.SemaphoreType.DMA((3,)),
            pltpu.SemaphoreType.DMA((6,)),
            pltpu.SemaphoreType.DMA((6,)),
            pltpu.VMEM_SHARED((N_PAD, D), jnp.float32),
        ],
    )
    return k(x, src, dst, edge_emb)


def _mlp_body(x_ref, p_ref, w1_ref, b1_ref, w2_ref, b2_ref, g_ref, bt_ref,
              eps_ref, out_ref):
    h = x_ref[...] * (1.0 + eps_ref[0, 0]) + p_ref[0] + p_ref[1]
    h1 = jnp.maximum(
        jnp.dot(h, w1_ref[...], preferred_element_type=jnp.float32)
        + b1_ref[...],
        0.0,
    )
    h2 = (
        jnp.dot(h1, w2_ref[...], preferred_element_type=jnp.float32)
        + b2_ref[...]
    )
    mean = jnp.mean(h2, axis=-1, keepdims=True)
    var = jnp.mean((h2 - mean) * (h2 - mean), axis=-1, keepdims=True)
    hn = (h2 - mean) * lax.rsqrt(var + 1e-5) * g_ref[...] + bt_ref[...]
    out_ref[...] = jnp.maximum(hn, 0.0)


def _mlp_ln(x, parts, W1, b1, W2, b2, gamma, beta, eps):
    blk_n = 1000
    grid = (N // blk_n,)
    return pl.pallas_call(
        _mlp_body,
        grid=grid,
        in_specs=[
            pl.BlockSpec((blk_n, D), lambda i: (i, 0)),
            pl.BlockSpec((NC, blk_n, D), lambda i: (0, i, 0)),
            pl.BlockSpec((D, D), lambda i: (0, 0)),
            pl.BlockSpec((1, D), lambda i: (0, 0)),
            pl.BlockSpec((D, D), lambda i: (0, 0)),
            pl.BlockSpec((1, D), lambda i: (0, 0)),
            pl.BlockSpec((1, D), lambda i: (0, 0)),
            pl.BlockSpec((1, D), lambda i: (0, 0)),
            pl.BlockSpec((1, 1), lambda i: (0, 0)),
        ],
        out_specs=pl.BlockSpec((blk_n, D), lambda i: (i, 0)),
        out_shape=jax.ShapeDtypeStruct((N, D), jnp.float32),
    )(x, parts, W1, b1, W2, b2, gamma, beta, eps)


def kernel(x, edge_index, edge_attr, batch, We, be, W1, b1, W2, b2, eps,
           gamma, beta):
    del batch
    edge_emb = _edge_emb(edge_attr.T, We, be.reshape(1, D))
    src = edge_index[0]
    dst = edge_index[1]
    parts = _sc_aggregate(x, src, dst, edge_emb)
    return _mlp_ln(
        x, parts, W1, b1.reshape(1, D), W2, b2.reshape(1, D),
        gamma.reshape(1, D), beta.reshape(1, D), eps.reshape(1, 1),
    )


# speedup vs baseline: 6.1709x; 1.0598x over previous
import functools

import jax
import jax.numpy as jnp
from jax import lax
from jax.experimental import pallas as pl
from jax.experimental.pallas import tpu as pltpu
from jax.experimental.pallas import tpu_sc as plsc

N = 10000
E = 320000
D = 128
ED = 16

NC = 2
NS = 16

BLK = 64
NH = 2
E_H = E // NH
NBLK_H = E_H // BLK
BLK_PER_SC = NBLK_H // NC
MAX_J = -(-BLK_PER_SC // NS)
N_PAD = 10240
ROWS_PER_TILE = N_PAD // NS


def _emb_body(ea_ref, we_ref, be_ref, out_ref):
    out_ref[...] = (
        lax.dot_general(
            ea_ref[...], we_ref[...], (((0,), (0,)), ((), ())),
            preferred_element_type=jnp.float32,
        )
        + be_ref[...]
    )


def _edge_emb(edge_attr_t, We, be, h):
    blk_e = 3200
    nblk = E_H // blk_e
    grid = (nblk,)
    return pl.pallas_call(
        _emb_body,
        grid=grid,
        in_specs=[
            pl.BlockSpec((ED, blk_e), lambda i: (0, h * nblk + i)),
            pl.BlockSpec((ED, D), lambda i: (0, 0)),
            pl.BlockSpec((1, D), lambda i: (0, 0)),
        ],
        out_specs=pl.BlockSpec((blk_e, D), lambda i: (i, 0)),
        out_shape=jax.ShapeDtypeStruct((E_H, D), jnp.float32),
    )(edge_attr_t, We, be)


def _sc_body(h, x_hbm, src_hbm, dst_hbm, emb_hbm, out_hbm,
             sidx, didx, emb_v, xrow_v, embsem, gsem, scsem, isem, dsem,
             acc_sh):
    c = lax.axis_index("c")
    s = lax.axis_index("s")

    def zero_row(r, _):
        for j in range(D // 16):
            emb_v[0, r, pl.ds(j * 16, 16)] = jnp.zeros((16,), jnp.float32)
        return 0

    lax.fori_loop(0, BLK, zero_row, 0)
    for t in range(ROWS_PER_TILE // BLK):
        pltpu.sync_copy(
            emb_v.at[0],
            acc_sh.at[pl.ds(s * ROWS_PER_TILE + t * BLK, BLK), :],
        )
    plsc.subcore_barrier()

    def valid(j):
        return j * NS + s < BLK_PER_SC

    def base_of(j):
        return (c * BLK_PER_SC + j * NS + s) * BLK

    def issue_idx(j, sl):
        base = h * E_H + base_of(j)
        pltpu.async_copy(src_hbm.at[pl.ds(base, BLK)], sidx.at[sl],
                         isem.at[sl])
        pltpu.async_copy(dst_hbm.at[pl.ds(base, BLK)], didx.at[sl, 0],
                         dsem.at[sl])

    def wait_idx(sl):
        pltpu.make_async_copy(src_hbm.at[pl.ds(0, BLK)], sidx.at[sl],
                              isem.at[sl]).wait()
        pltpu.make_async_copy(dst_hbm.at[pl.ds(0, BLK)], didx.at[sl, 0],
                              dsem.at[sl]).wait()

    def issue_emb(j, e):
        pltpu.async_copy(emb_hbm.at[pl.ds(base_of(j), BLK), :],
                         emb_v.at[e], embsem.at[e])

    def wait_emb(e):
        pltpu.make_async_copy(emb_hbm.at[pl.ds(0, BLK), :], emb_v.at[e],
                              embsem.at[e]).wait()

    def issue_gather(sl, xs):
        pltpu.async_copy(x_hbm.at[sidx.at[sl]], xrow_v.at[xs], gsem.at[xs])

    def wait_gather(xs):
        pltpu.make_async_copy(x_hbm.at[sidx.at[0]], xrow_v.at[xs],
                              gsem.at[xs]).wait()

    def issue_scatter(e, sl):
        pltpu.make_async_copy(emb_v.at[e], acc_sh.at[didx.at[sl, 0]],
                              scsem.at[e]).start(add=True)

    def wait_scatter(e):
        pltpu.make_async_copy(emb_v.at[e], acc_sh.at[didx.at[0, 0]],
                              scsem.at[e]).wait()

    def compute(e, xs):
        def ew(r2, _2):
            for u in range(4):
                r = 4 * r2 + u
                for g in range(D // 16):
                    sl = pl.ds(g * 16, 16)
                    emb_v[e, r, sl] = jnp.maximum(
                        emb_v[e, r, sl] + xrow_v[xs, r, sl], 0.0
                    )
            return 0

        lax.fori_loop(0, BLK // 4, ew, 0)

    for k in range(6):
        issue_idx(k, k)
    for e in range(3):
        issue_emb(e, e)
    for j0 in range(2):
        wait_idx(j0)
        issue_gather(j0, j0)
    for j0 in range(2):
        wait_emb(j0 % 3)
        wait_gather(j0 % 2)
        compute(j0 % 3, j0 % 2)
        issue_scatter(j0 % 3, j0 % 6)
        wait_idx(j0 + 2)
        issue_gather(j0 + 2, j0 % 2)

    def step(q, _):
        for u in range(6):
            j = 2 + 6 * q + u
            e3 = (2 + u) % 3
            x2 = u % 2
            i6 = (2 + u) % 6

            @pl.when(valid(j + 1))
            def _():
                wait_scatter((e3 + 1) % 3)
                issue_emb(j + 1, (e3 + 1) % 3)

            @pl.when(valid(j + 4))
            def _():
                issue_idx(j + 4, (i6 + 4) % 6)

            @pl.when(valid(j))
            def _():
                wait_emb(e3)
                wait_gather(x2)
                compute(e3, x2)
                issue_scatter(e3, i6)

            @pl.when(valid(j + 2))
            def _():
                wait_idx((i6 + 2) % 6)
                issue_gather((i6 + 2) % 6, x2)

        return 0

    lax.fori_loop(0, (MAX_J - 2 + 5) // 6, step, 0)

    for e in range(3):
        wait_scatter(e)
    plsc.subcore_barrier()

    for t in range(ROWS_PER_TILE // BLK):
        row0 = s * ROWS_PER_TILE + t * BLK
        pltpu.sync_copy(acc_sh.at[pl.ds(row0, BLK), :], emb_v.at[0])
        pltpu.sync_copy(emb_v.at[0], out_hbm.at[c, pl.ds(row0, BLK), :])


def _sc_aggregate(x, src, dst, edge_emb, h):
    mesh = plsc.VectorSubcoreMesh(core_axis_name="c", subcore_axis_name="s")
    k = pl.kernel(
        functools.partial(_sc_body, h),
        out_type=jax.ShapeDtypeStruct((NC, N_PAD, D), jnp.float32),
        mesh=mesh,
        scratch_types=[
            pltpu.VMEM((6, BLK), jnp.int32),
            pltpu.VMEM((6, 1, BLK), jnp.int32),
            pltpu.VMEM((3, BLK, D), jnp.float32),
            pltpu.VMEM((2, BLK, D), jnp.float32),
            pltpu.SemaphoreType.DMA((3,)),
            pltpu.SemaphoreType.DMA((2,)),
            pltpu.SemaphoreType.DMA((3,)),
            pltpu.SemaphoreType.DMA((6,)),
            pltpu.SemaphoreType.DMA((6,)),
            pltpu.VMEM_SHARED((N_PAD, D), jnp.float32),
        ],
    )
    return k(x, src, dst, edge_emb)


def _mlp_body(x_ref, p_ref, q_ref, w1_ref, b1_ref, w2_ref, b2_ref, g_ref,
              bt_ref, eps_ref, out_ref):
    h = (x_ref[...] * (1.0 + eps_ref[0, 0]) + p_ref[0] + p_ref[1]
         + q_ref[0] + q_ref[1])
    h1 = jnp.maximum(
        jnp.dot(h, w1_ref[...], preferred_element_type=jnp.float32)
        + b1_ref[...],
        0.0,
    )
    h2 = (
        jnp.dot(h1, w2_ref[...], preferred_element_type=jnp.float32)
        + b2_ref[...]
    )
    mean = jnp.mean(h2, axis=-1, keepdims=True)
    var = jnp.mean((h2 - mean) * (h2 - mean), axis=-1, keepdims=True)
    hn = (h2 - mean) * lax.rsqrt(var + 1e-5) * g_ref[...] + bt_ref[...]
    out_ref[...] = jnp.maximum(hn, 0.0)


def _mlp_ln(x, parts0, parts1, W1, b1, W2, b2, gamma, beta, eps):
    blk_n = 1000
    grid = (N // blk_n,)
    return pl.pallas_call(
        _mlp_body,
        grid=grid,
        in_specs=[
            pl.BlockSpec((blk_n, D), lambda i: (i, 0)),
            pl.BlockSpec((NC, blk_n, D), lambda i: (0, i, 0)),
            pl.BlockSpec((NC, blk_n, D), lambda i: (0, i, 0)),
            pl.BlockSpec((D, D), lambda i: (0, 0)),
            pl.BlockSpec((1, D), lambda i: (0, 0)),
            pl.BlockSpec((D, D), lambda i: (0, 0)),
            pl.BlockSpec((1, D), lambda i: (0, 0)),
            pl.BlockSpec((1, D), lambda i: (0, 0)),
            pl.BlockSpec((1, D), lambda i: (0, 0)),
            pl.BlockSpec((1, 1), lambda i: (0, 0)),
        ],
        out_specs=pl.BlockSpec((blk_n, D), lambda i: (i, 0)),
        out_shape=jax.ShapeDtypeStruct((N, D), jnp.float32),
    )(x, parts0, parts1, W1, b1, W2, b2, gamma, beta, eps)


def kernel(x, edge_index, edge_attr, batch, We, be, W1, b1, W2, b2, eps,
           gamma, beta):
    del batch
    ea_t = edge_attr.T
    src = edge_index[0]
    dst = edge_index[1]
    emb0 = _edge_emb(ea_t, We, be.reshape(1, D), 0)
    emb1 = _edge_emb(ea_t, We, be.reshape(1, D), 1)
    parts0 = _sc_aggregate(x, src, dst, emb0, 0)
    parts1 = _sc_aggregate(x, src, dst, emb1, 1)
    return _mlp_ln(
        x, parts0, parts1, W1, b1.reshape(1, D), W2, b2.reshape(1, D),
        gamma.reshape(1, D), beta.reshape(1, D), eps.reshape(1, 1),
    )


# speedup vs baseline: 6.5660x; 1.0640x over previous
import functools

import jax
import jax.numpy as jnp
from jax import lax
from jax.experimental import pallas as pl
from jax.experimental.pallas import tpu as pltpu
from jax.experimental.pallas import tpu_sc as plsc

N = 10000
E = 320000
D = 128
ED = 16

NC = 2
NS = 16

BLK = 64
NH = 2
E_H = E // NH
NBLK_H = E_H // BLK
BLK_PER_SC = NBLK_H // NC
MAX_J = -(-BLK_PER_SC // NS)
N_PAD = 10240
ROWS_PER_TILE = N_PAD // NS


def _emb_body(ea_ref, we_ref, be_ref, out_ref):
    out_ref[...] = (
        lax.dot_general(
            ea_ref[...], we_ref[...], (((0,), (0,)), ((), ())),
            preferred_element_type=jnp.float32,
        )
        + be_ref[...]
    )


def _edge_emb(edge_attr_t, We, be, h):
    blk_e = 3200
    nblk = E_H // blk_e
    grid = (nblk,)
    return pl.pallas_call(
        _emb_body,
        grid=grid,
        in_specs=[
            pl.BlockSpec((ED, blk_e), lambda i: (0, h * nblk + i)),
            pl.BlockSpec((ED, D), lambda i: (0, 0)),
            pl.BlockSpec((1, D), lambda i: (0, 0)),
        ],
        out_specs=pl.BlockSpec((blk_e, D), lambda i: (i, 0)),
        out_shape=jax.ShapeDtypeStruct((E_H, D), jnp.float32),
    )(edge_attr_t, We, be)


def _sc_body(h, x_hbm, ei_hbm, emb_hbm, out_hbm,
             sidx, didx, emb_v, xrow_v, embsem, gsem, scsem, isem, dsem,
             acc_sh):
    c = lax.axis_index("c")
    s = lax.axis_index("s")

    def zero_row(r, _):
        for j in range(D // 16):
            emb_v[0, r, pl.ds(j * 16, 16)] = jnp.zeros((16,), jnp.float32)
        return 0

    lax.fori_loop(0, BLK, zero_row, 0)
    for t in range(ROWS_PER_TILE // BLK):
        pltpu.async_copy(
            emb_v.at[0],
            acc_sh.at[pl.ds(s * ROWS_PER_TILE + t * BLK, BLK), :],
            scsem.at[0],
        )
    for t in range(ROWS_PER_TILE // BLK):
        pltpu.make_async_copy(
            emb_v.at[0],
            acc_sh.at[pl.ds(s * ROWS_PER_TILE, BLK), :],
            scsem.at[0],
        ).wait()
    plsc.subcore_barrier()

    def valid(j):
        return j * NS + s < BLK_PER_SC

    def base_of(j):
        return (c * BLK_PER_SC + j * NS + s) * BLK

    def issue_idx(j, sl):
        base = h * E_H + base_of(j)
        pltpu.async_copy(ei_hbm.at[0, pl.ds(base, BLK)], sidx.at[sl],
                         isem.at[sl])
        pltpu.async_copy(ei_hbm.at[1, pl.ds(base, BLK)], didx.at[sl, 0],
                         dsem.at[sl])

    def wait_idx(sl):
        pltpu.make_async_copy(ei_hbm.at[0, pl.ds(0, BLK)], sidx.at[sl],
                              isem.at[sl]).wait()
        pltpu.make_async_copy(ei_hbm.at[1, pl.ds(0, BLK)], didx.at[sl, 0],
                              dsem.at[sl]).wait()

    def issue_emb(j, e):
        pltpu.async_copy(emb_hbm.at[pl.ds(base_of(j), BLK), :],
                         emb_v.at[e], embsem.at[e])

    def wait_emb(e):
        pltpu.make_async_copy(emb_hbm.at[pl.ds(0, BLK), :], emb_v.at[e],
                              embsem.at[e]).wait()

    def issue_gather(sl, xs):
        pltpu.async_copy(x_hbm.at[sidx.at[sl]], xrow_v.at[xs], gsem.at[xs])

    def wait_gather(xs):
        pltpu.make_async_copy(x_hbm.at[sidx.at[0]], xrow_v.at[xs],
                              gsem.at[xs]).wait()

    def issue_scatter(e, sl):
        pltpu.make_async_copy(emb_v.at[e], acc_sh.at[didx.at[sl, 0]],
                              scsem.at[e]).start(add=True)

    def wait_scatter(e):
        pltpu.make_async_copy(emb_v.at[e], acc_sh.at[didx.at[0, 0]],
                              scsem.at[e]).wait()

    def compute(e, xs):
        def ew(r2, _2):
            for u in range(4):
                r = 4 * r2 + u
                for g in range(D // 16):
                    sl = pl.ds(g * 16, 16)
                    emb_v[e, r, sl] = jnp.maximum(
                        emb_v[e, r, sl] + xrow_v[xs, r, sl], 0.0
                    )
            return 0

        lax.fori_loop(0, BLK // 4, ew, 0)

    for k in range(6):
        issue_idx(k, k)
    for e in range(3):
        issue_emb(e, e)
    for j0 in range(2):
        wait_idx(j0)
        issue_gather(j0, j0)
    for j0 in range(2):
        wait_emb(j0 % 3)
        wait_gather(j0 % 2)
        compute(j0 % 3, j0 % 2)
        issue_scatter(j0 % 3, j0 % 6)
        wait_idx(j0 + 2)
        issue_gather(j0 + 2, j0 % 2)

    def step(q, _):
        for u in range(6):
            j = 2 + 6 * q + u
            e3 = (2 + u) % 3
            x2 = u % 2
            i6 = (2 + u) % 6

            @pl.when(valid(j + 1))
            def _():
                wait_scatter((e3 + 1) % 3)
                issue_emb(j + 1, (e3 + 1) % 3)

            @pl.when(valid(j + 4))
            def _():
                issue_idx(j + 4, (i6 + 4) % 6)

            @pl.when(valid(j))
            def _():
                wait_emb(e3)
                wait_gather(x2)
                compute(e3, x2)
                issue_scatter(e3, i6)

            @pl.when(valid(j + 2))
            def _():
                wait_idx((i6 + 2) % 6)
                issue_gather((i6 + 2) % 6, x2)

        return 0

    lax.fori_loop(0, (MAX_J - 2 + 5) // 6, step, 0)

    for e in range(3):
        wait_scatter(e)
    plsc.subcore_barrier()

    row0 = s * ROWS_PER_TILE
    pltpu.sync_copy(
        acc_sh.at[pl.ds(row0, ROWS_PER_TILE), :],
        out_hbm.at[c, pl.ds(row0, ROWS_PER_TILE), :],
    )


def _sc_aggregate(x, edge_index, edge_emb, h):
    mesh = plsc.VectorSubcoreMesh(core_axis_name="c", subcore_axis_name="s")
    k = pl.kernel(
        functools.partial(_sc_body, h),
        out_type=jax.ShapeDtypeStruct((NC, N_PAD, D), jnp.float32),
        mesh=mesh,
        scratch_types=[
            pltpu.VMEM((6, BLK), jnp.int32),
            pltpu.VMEM((6, 1, BLK), jnp.int32),
            pltpu.VMEM((3, BLK, D), jnp.float32),
            pltpu.VMEM((2, BLK, D), jnp.float32),
            pltpu.SemaphoreType.DMA((3,)),
            pltpu.SemaphoreType.DMA((2,)),
            pltpu.SemaphoreType.DMA((3,)),
            pltpu.SemaphoreType.DMA((6,)),
            pltpu.SemaphoreType.DMA((6,)),
            pltpu.VMEM_SHARED((N_PAD, D), jnp.float32),
        ],
    )
    return k(x, edge_index, edge_emb)


def _mlp_body(x_ref, p_ref, q_ref, w1_ref, b1_ref, w2_ref, b2_ref, g_ref,
              bt_ref, eps_ref, out_ref):
    h = (x_ref[...] * (1.0 + eps_ref[0, 0]) + p_ref[0] + p_ref[1]
         + q_ref[0] + q_ref[1])
    h1 = jnp.maximum(
        jnp.dot(h, w1_ref[...], preferred_element_type=jnp.float32)
        + b1_ref[...],
        0.0,
    )
    h2 = (
        jnp.dot(h1, w2_ref[...], preferred_element_type=jnp.float32)
        + b2_ref[...]
    )
    mean = jnp.mean(h2, axis=-1, keepdims=True)
    var = jnp.mean((h2 - mean) * (h2 - mean), axis=-1, keepdims=True)
    hn = (h2 - mean) * lax.rsqrt(var + 1e-5) * g_ref[...] + bt_ref[...]
    out_ref[...] = jnp.maximum(hn, 0.0)


def _mlp_ln(x, parts0, parts1, W1, b1, W2, b2, gamma, beta, eps):
    blk_n = 1000
    grid = (N // blk_n,)
    return pl.pallas_call(
        _mlp_body,
        grid=grid,
        in_specs=[
            pl.BlockSpec((blk_n, D), lambda i: (i, 0)),
            pl.BlockSpec((NC, blk_n, D), lambda i: (0, i, 0)),
            pl.BlockSpec((NC, blk_n, D), lambda i: (0, i, 0)),
            pl.BlockSpec((D, D), lambda i: (0, 0)),
            pl.BlockSpec((1, D), lambda i: (0, 0)),
            pl.BlockSpec((D, D), lambda i: (0, 0)),
            pl.BlockSpec((1, D), lambda i: (0, 0)),
            pl.BlockSpec((1, D), lambda i: (0, 0)),
            pl.BlockSpec((1, D), lambda i: (0, 0)),
            pl.BlockSpec((1, 1), lambda i: (0, 0)),
        ],
        out_specs=pl.BlockSpec((blk_n, D), lambda i: (i, 0)),
        out_shape=jax.ShapeDtypeStruct((N, D), jnp.float32),
    )(x, parts0, parts1, W1, b1, W2, b2, gamma, beta, eps)


def kernel(x, edge_index, edge_attr, batch, We, be, W1, b1, W2, b2, eps,
           gamma, beta):
    del batch
    ea_t = edge_attr.T
    emb0 = _edge_emb(ea_t, We, be.reshape(1, D), 0)
    emb1 = _edge_emb(ea_t, We, be.reshape(1, D), 1)
    parts0 = _sc_aggregate(x, edge_index, emb0, 0)
    parts1 = _sc_aggregate(x, edge_index, emb1, 1)
    return _mlp_ln(
        x, parts0, parts1, W1, b1.reshape(1, D), W2, b2.reshape(1, D),
        gamma.reshape(1, D), beta.reshape(1, D), eps.reshape(1, 1),
    )
